# Initial kernel scaffold; baseline (speedup 1.0000x reference)
import jax
import jax.numpy as jnp
from jax.experimental import pallas as pl


def kernel(token_ids, embeddings):
    raise NotImplementedError("write your pallas kernel here")


# speedup vs baseline: 1.8778x; 1.8778x over previous
import functools

import jax
import jax.numpy as jnp
from jax import lax
from jax.experimental import pallas as pl
from jax.experimental.pallas import tpu as pltpu
from jax.experimental.pallas import tpu_sc as plsc

NUM_CORES = 2
NUM_SUBCORES = 16
NUM_WORKERS = NUM_CORES * NUM_SUBCORES

CHUNK = 128
CHUNKS_PER_GROUP = 4
GROUP = CHUNK * CHUNKS_PER_GROUP


def _sc_gather(table, idx2d, *, n_rows, dim):
    rows_per_w = n_rows // NUM_WORKERS
    groups_per_w = rows_per_w // GROUP
    chunk_rows_per_w = rows_per_w // CHUNK
    n_body = groups_per_w // 2 - 1

    mesh = plsc.VectorSubcoreMesh(core_axis_name="c", subcore_axis_name="s")

    @functools.partial(
        pl.kernel,
        mesh=mesh,
        out_type=jax.ShapeDtypeStruct((n_rows, dim), jnp.float32),
        compiler_params=pltpu.CompilerParams(use_tc_tiling_on_sc=False),
        scratch_types=[
            pltpu.VMEM((chunk_rows_per_w, CHUNK), jnp.int32),
            pltpu.VMEM((GROUP, dim), jnp.float32),
            pltpu.VMEM((GROUP, dim), jnp.float32),
            pltpu.SemaphoreType.DMA,
            pltpu.SemaphoreType.DMA,
            pltpu.SemaphoreType.DMA,
            pltpu.SemaphoreType.DMA,
        ],
    )
    def k(table_hbm, idx_hbm, out_hbm, idx_v, buf_a, buf_b,
          gsem_a, gsem_b, ssem_a, ssem_b):
        wid = lax.axis_index("s") * NUM_CORES + lax.axis_index("c")
        out_base = wid * rows_per_w

        pltpu.sync_copy(idx_hbm.at[pl.ds(wid * chunk_rows_per_w,
                                         chunk_rows_per_w)], idx_v)

        def fire_gathers(buf, gsem, s):
            for b in range(CHUNKS_PER_GROUP):
                pltpu.async_copy(
                    table_hbm.at[idx_v.at[s * CHUNKS_PER_GROUP + b]],
                    buf.at[pl.ds(b * CHUNK, CHUNK)],
                    gsem)

        def wait_gathers(buf, gsem):
            pltpu.make_async_copy(table_hbm.at[pl.ds(0, GROUP)], buf,
                                  gsem).wait()

        def fire_store(buf, ssem, s):
            pltpu.async_copy(buf, out_hbm.at[pl.ds(out_base + s * GROUP,
                                                   GROUP)], ssem)

        def wait_store(buf, ssem, s):
            pltpu.make_async_copy(buf, out_hbm.at[pl.ds(out_base + s * GROUP,
                                                        GROUP)], ssem).wait()

        fire_gathers(buf_a, gsem_a, 0)
        fire_gathers(buf_b, gsem_b, 1)

        def body(t, _):
            s_a = 2 * t
            s_b = s_a + 1
            wait_gathers(buf_a, gsem_a)
            fire_store(buf_a, ssem_a, s_a)
            wait_store(buf_a, ssem_a, s_a)
            fire_gathers(buf_a, gsem_a, s_a + 2)
            wait_gathers(buf_b, gsem_b)
            fire_store(buf_b, ssem_b, s_b)
            wait_store(buf_b, ssem_b, s_b)
            fire_gathers(buf_b, gsem_b, s_b + 2)
            return _

        lax.fori_loop(0, n_body, body, 0)

        s_last = groups_per_w - 2
        wait_gathers(buf_a, gsem_a)
        fire_store(buf_a, ssem_a, s_last)
        wait_store(buf_a, ssem_a, s_last)
        wait_gathers(buf_b, gsem_b)
        fire_store(buf_b, ssem_b, s_last + 1)
        wait_store(buf_b, ssem_b, s_last + 1)

    return k(table, idx2d)


def kernel(token_ids, embeddings):
    n_tok, seq = token_ids.shape
    dim = embeddings.shape[1]
    n_rows = n_tok * seq
    idx2d = token_ids.astype(jnp.int32).reshape(n_rows // CHUNK, CHUNK)
    out = _sc_gather(embeddings, idx2d, n_rows=n_rows, dim=dim)
    return out.reshape(n_tok, seq, dim)
